# baseline (speedup 1.0000x reference)
import functools

import jax
import jax.numpy as jnp
from jax import lax
from jax.experimental import pallas as pl
from jax.experimental.pallas import tpu as pltpu
from jax.experimental.pallas import tpu_sc as plsc

B = 4096
BLK = 512
NBLK = B // BLK
NW = 32
CHUNK = B // NW


def _dense_kernel(sensor_s_ref, sensor_ref, other_s_ref, other_ref,
                  out_ref, acc_ref):
    i = pl.program_id(0)

    @pl.when(i == 0)
    def _init():
        acc_ref[0] = 0.0
        acc_ref[1] = 0.0

    acc_ref[0] += jnp.sum(jnp.abs(sensor_s_ref[...] - sensor_ref[...]))
    acc_ref[1] += jnp.sum(jnp.abs(other_s_ref[...] - other_ref[...]))

    @pl.when(i == NBLK - 1)
    def _final():
        out_ref[0] = acc_ref[0]
        out_ref[1] = acc_ref[1]


def _ber_kernel(apt_ref, spt_ref, act_ref, sens_ref, out_ref,
                act_v, sens_v, idxa_v, idxs_v, ga_v, gs_v, acc_v, sem):
    c = lax.axis_index("c")
    s = lax.axis_index("s")
    wid = s * 2 + c
    base = wid * CHUNK

    pltpu.sync_copy(act_ref.at[pl.ds(base, CHUNK)], act_v)
    pltpu.sync_copy(sens_ref.at[pl.ds(base, CHUNK)], sens_v)

    io16 = lax.iota(jnp.int32, 16)
    for j in range(CHUNK // 16):
        pos = base + j * 16 + io16
        idxa_v[pl.ds(j * 16, 16)] = pos * 12 + act_v[pl.ds(j * 16, 16)]
        idxs_v[pl.ds(j * 16, 16)] = pos * 4 + sens_v[pl.ds(j * 16, 16)]

    ca = pltpu.async_copy(apt_ref.at[idxa_v], ga_v, sem)
    cs = pltpu.async_copy(spt_ref.at[idxs_v], gs_v, sem)
    ca.wait()
    cs.wait()

    zeros = jnp.zeros((16,), jnp.float32)
    ones = jnp.full((16,), 1.0, jnp.float32)
    acc = [zeros] * 12
    for j in range(CHUNK // 16):
        sg = sens_v[pl.ds(j * 16, 16)]
        va = jnp.abs(1.0 - ga_v[pl.ds(j * 16, 16)])
        vs = jnp.abs(1.0 - gs_v[pl.ds(j * 16, 16)])
        for g in range(4):
            m = sg == g
            acc[g] = acc[g] + jnp.where(m, va, zeros)
            acc[4 + g] = acc[4 + g] + jnp.where(m, vs, zeros)
            acc[8 + g] = acc[8 + g] + jnp.where(m, ones, zeros)

    for g in range(12):
        acc_v[pl.ds(g * 16, 16)] = acc[g]
    pltpu.sync_copy(acc_v, out_ref.at[wid])


_ber_call = pl.kernel(
    _ber_kernel,
    mesh=plsc.VectorSubcoreMesh(core_axis_name="c", subcore_axis_name="s"),
    out_type=jax.ShapeDtypeStruct((NW, 192), jnp.float32),
    scratch_types=[
        pltpu.VMEM((CHUNK,), jnp.int32),
        pltpu.VMEM((CHUNK,), jnp.int32),
        pltpu.VMEM((CHUNK,), jnp.int32),
        pltpu.VMEM((CHUNK,), jnp.int32),
        pltpu.VMEM((CHUNK,), jnp.float32),
        pltpu.VMEM((CHUNK,), jnp.float32),
        pltpu.VMEM((192,), jnp.float32),
        pltpu.SemaphoreType.DMA,
    ],
)


def kernel(sensor_s, other_s, act_p, sens_p, sensor, act, sens, other):
    c, t = sensor_s.shape[1], sensor_s.shape[2]
    st = jnp.transpose(sensor_s, (1, 0, 2))
    rt = jnp.transpose(sensor, (1, 0, 2))
    ot_s = other_s.T
    ot = other.T

    apt1 = act_p.reshape(B * act_p.shape[1])
    spt1 = sens_p.reshape(B * sens_p.shape[1])
    parts = _ber_call(apt1, spt1, act, sens)

    dense = pl.pallas_call(
        _dense_kernel,
        grid=(NBLK,),
        in_specs=[
            pl.BlockSpec((c, BLK, t), lambda i: (0, i, 0)),
            pl.BlockSpec((c, BLK, t), lambda i: (0, i, 0)),
            pl.BlockSpec((ot_s.shape[0], BLK), lambda i: (0, i)),
            pl.BlockSpec((ot.shape[0], BLK), lambda i: (0, i)),
        ],
        out_specs=pl.BlockSpec(memory_space=pltpu.SMEM),
        out_shape=jax.ShapeDtypeStruct((2,), jnp.float32),
        scratch_shapes=[pltpu.SMEM((2,), jnp.float32)],
    )(st, rt, ot_s, ot)

    tot = jnp.sum(parts.reshape(NW, 12, 16), axis=(0, 2))
    sa, ss, cnt = tot[0:4], tot[4:8], tot[8:12]
    n_groups = jnp.max(jnp.where(cnt > 0, jnp.arange(1.0, 5.0), 0.0))
    safe = jnp.maximum(cnt, 1e-12)
    act_loss = jnp.abs(0.0 - jnp.sum(sa / safe) / n_groups)
    sens_loss = jnp.abs(0.5 - jnp.sum(ss / safe) / n_groups)
    sensor_loss = dense[0] / (B * 6.0 * 512.0)
    physio_loss = dense[1] / (B * 16.0)
    combined = (0.25 * act_loss + 0.25 * sens_loss
                + 0.5 * 0.5 * (sensor_loss + physio_loss))
    return (combined, act_loss, sens_loss)


# speedup vs baseline: 1.0832x; 1.0832x over previous
import functools

import jax
import jax.numpy as jnp
from jax import lax
from jax.experimental import pallas as pl
from jax.experimental.pallas import tpu as pltpu
from jax.experimental.pallas import tpu_sc as plsc

B = 4096
BLK = 512
NBLK = B // BLK
NW = 32
CHUNK = B // NW


def _dense_kernel(sensor_s_ref, sensor_ref, other_s_ref, other_ref,
                  out_ref, acc_ref):
    i = pl.program_id(0)

    @pl.when(i == 0)
    def _init():
        acc_ref[0] = 0.0
        acc_ref[1] = 0.0

    acc_ref[0] += jnp.sum(jnp.abs(sensor_s_ref[...] - sensor_ref[...]))
    acc_ref[1] += jnp.sum(jnp.abs(other_s_ref[...] - other_ref[...]))

    @pl.when(i == NBLK - 1)
    def _final():
        out_ref[0] = acc_ref[0]
        out_ref[1] = acc_ref[1]


def _ber_kernel(apt_ref, spt_ref, act_ref, sens_ref, out_ref,
                act_v, sens_v, idxa_v, idxs_v, ga_v, gs_v, acc_v, sem):
    c = lax.axis_index("c")
    s = lax.axis_index("s")
    wid = s * 2 + c
    base = wid * CHUNK

    pltpu.sync_copy(act_ref.at[pl.ds(base, CHUNK)], act_v)
    pltpu.sync_copy(sens_ref.at[pl.ds(base, CHUNK)], sens_v)

    io16 = lax.iota(jnp.int32, 16)
    for j in range(CHUNK // 16):
        pos = base + j * 16 + io16
        idxa_v[pl.ds(j * 16, 16)] = act_v[pl.ds(j * 16, 16)] * B + pos
        idxs_v[pl.ds(j * 16, 16)] = sens_v[pl.ds(j * 16, 16)] * B + pos

    ca = pltpu.async_copy(apt_ref.at[idxa_v], ga_v, sem)
    cs = pltpu.async_copy(spt_ref.at[idxs_v], gs_v, sem)
    ca.wait()
    cs.wait()

    zeros = jnp.zeros((16,), jnp.float32)
    ones = jnp.full((16,), 1.0, jnp.float32)
    acc = [zeros] * 12
    for j in range(CHUNK // 16):
        sg = sens_v[pl.ds(j * 16, 16)]
        va = jnp.abs(1.0 - ga_v[pl.ds(j * 16, 16)])
        vs = jnp.abs(1.0 - gs_v[pl.ds(j * 16, 16)])
        for g in range(4):
            m = sg == g
            acc[g] = acc[g] + jnp.where(m, va, zeros)
            acc[4 + g] = acc[4 + g] + jnp.where(m, vs, zeros)
            acc[8 + g] = acc[8 + g] + jnp.where(m, ones, zeros)

    for g in range(12):
        acc_v[pl.ds(g * 16, 16)] = acc[g]
    pltpu.sync_copy(acc_v, out_ref.at[wid])


_ber_call = pl.kernel(
    _ber_kernel,
    mesh=plsc.VectorSubcoreMesh(core_axis_name="c", subcore_axis_name="s"),
    out_type=jax.ShapeDtypeStruct((NW, 192), jnp.float32),
    scratch_types=[
        pltpu.VMEM((CHUNK,), jnp.int32),
        pltpu.VMEM((CHUNK,), jnp.int32),
        pltpu.VMEM((CHUNK,), jnp.int32),
        pltpu.VMEM((CHUNK,), jnp.int32),
        pltpu.VMEM((CHUNK,), jnp.float32),
        pltpu.VMEM((CHUNK,), jnp.float32),
        pltpu.VMEM((192,), jnp.float32),
        pltpu.SemaphoreType.DMA,
    ],
)


def kernel(sensor_s, other_s, act_p, sens_p, sensor, act, sens, other):
    c, t = sensor_s.shape[1], sensor_s.shape[2]
    st = jnp.transpose(sensor_s, (1, 0, 2))
    rt = jnp.transpose(sensor, (1, 0, 2))
    ot_s = other_s.T
    ot = other.T

    apt1 = act_p.T.reshape(B * act_p.shape[1])
    spt1 = sens_p.T.reshape(B * sens_p.shape[1])
    parts = _ber_call(apt1, spt1, act, sens)

    dense = pl.pallas_call(
        _dense_kernel,
        grid=(NBLK,),
        in_specs=[
            pl.BlockSpec((c, BLK, t), lambda i: (0, i, 0)),
            pl.BlockSpec((c, BLK, t), lambda i: (0, i, 0)),
            pl.BlockSpec((ot_s.shape[0], BLK), lambda i: (0, i)),
            pl.BlockSpec((ot.shape[0], BLK), lambda i: (0, i)),
        ],
        out_specs=pl.BlockSpec(memory_space=pltpu.SMEM),
        out_shape=jax.ShapeDtypeStruct((2,), jnp.float32),
        scratch_shapes=[pltpu.SMEM((2,), jnp.float32)],
    )(st, rt, ot_s, ot)

    tot = jnp.sum(parts.reshape(NW, 12, 16), axis=(0, 2))
    sa, ss, cnt = tot[0:4], tot[4:8], tot[8:12]
    n_groups = jnp.max(jnp.where(cnt > 0, jnp.arange(1.0, 5.0), 0.0))
    safe = jnp.maximum(cnt, 1e-12)
    act_loss = jnp.abs(0.0 - jnp.sum(sa / safe) / n_groups)
    sens_loss = jnp.abs(0.5 - jnp.sum(ss / safe) / n_groups)
    sensor_loss = dense[0] / (B * 6.0 * 512.0)
    physio_loss = dense[1] / (B * 16.0)
    combined = (0.25 * act_loss + 0.25 * sens_loss
                + 0.5 * 0.5 * (sensor_loss + physio_loss))
    return (combined, act_loss, sens_loss)
